# baseline (speedup 1.0000x reference)
import functools

import jax
import jax.numpy as jnp
from jax import lax
from jax.experimental import pallas as pl
from jax.experimental.pallas import tpu as pltpu
from jax.experimental.pallas import tpu_sc as plsc

N = 10000
NF = 1433
V = 16
D = 32
FEAT = 31
H = 2
OUT = 7
E = 160000

NP = 10240
NFP = 1536
RB = 256
AB = 32
ABV = AB * V
DH = D // H
NEG = -1e30

NTILES = 16
NROWS = NP // NTILES
EPT = E // NTILES
CHUNK = 125
ZCH = 128


def _colsum_body(x_ref, out_ref):
    @pl.when(pl.program_id(0) == 0)
    def _init():
        out_ref[...] = jnp.zeros_like(out_ref)

    out_ref[...] += x_ref[...].reshape(RB // 8, 8, NFP).sum(axis=0)


def _colsum(xp):
    return pl.pallas_call(
        _colsum_body,
        grid=(NP // RB,),
        in_specs=[pl.BlockSpec((RB, NFP), lambda i: (i, 0))],
        out_specs=pl.BlockSpec((8, NFP), lambda i: (0, 0)),
        out_shape=jax.ShapeDtypeStruct((8, NFP), jnp.float32),
    )(xp)


def _tokens_body(x_ref, g_ref, cnt_ref, emb_ref, out_ref):
    xb = x_ref[...]
    gb = g_ref[...]

    cntc = cnt_ref[:, 0:1]
    p = cntc * (1.0 / N)
    sd = jnp.sqrt(p * (1.0 - p))
    sd1 = jnp.where(sd == 0.0, 1.0, sd)
    a0 = (0.0 - p) / sd1
    dd = 1.0 / sd1
    lane_t = lax.broadcasted_iota(jnp.int32, (NFP, 128), 1)
    embx = jnp.where(lane_t == FEAT, a0,
                     jnp.where(lane_t == FEAT + 1, dd, emb_ref[...]))
    embx_bf = embx.astype(jnp.bfloat16)

    npres = xb.sum(axis=1, keepdims=True)
    cpres = jnp.log(0.5 / jnp.maximum(npres, 1.0))
    cabs = jnp.log(0.5 / jnp.maximum(NF - npres, 1.0))
    scores = gb + jnp.where(xb != 0.0, cpres, cabs)

    colid = lax.broadcasted_iota(jnp.int32, (RB, NFP), 1)
    lane_o = lax.broadcasted_iota(jnp.int32, (RB, 128), 1)
    for t in range(V):
        m = scores.max(axis=1, keepdims=True)
        sc = jnp.min(jnp.where(scores == m, colid, NFP), axis=1, keepdims=True)
        selb = colid == sc
        sel = selb.astype(jnp.float32)
        row = lax.dot(selb.astype(jnp.bfloat16), embx_bf,
                      preferred_element_type=jnp.float32)
        pres = (sel * xb).sum(axis=1, keepdims=True)
        val = row[:, FEAT:FEAT + 1] + pres * row[:, FEAT + 1:FEAT + 2]
        tok = jnp.where(lane_o == FEAT, val, row)
        out_ref[t // 4, :, (t % 4) * D:(t % 4) * D + D] = tok[:, :D]
        scores = jnp.where(selb, NEG, scores)


def _tokens(xp, gp, cntT, embp):
    return pl.pallas_call(
        _tokens_body,
        grid=(NP // RB,),
        in_specs=[
            pl.BlockSpec((RB, NFP), lambda i: (i, 0)),
            pl.BlockSpec((RB, NFP), lambda i: (i, 0)),
            pl.BlockSpec((NFP, 128), lambda i: (0, 0)),
            pl.BlockSpec((NFP, 128), lambda i: (0, 0)),
        ],
        out_specs=pl.BlockSpec((4, RB, 128), lambda i: (0, i, 0)),
        out_shape=jax.ShapeDtypeStruct((4, NP, 128), jnp.float32),
    )(xp, gp, cntT, embp)


NCH = EPT // CHUNK
NB = 5
NGRP = NCH // NB


def _agg_body(tok_hbm, src0_hbm, src1_hbm, dst_hbm, zrow_hbm, z16_hbm,
              ones16_hbm, agg_hbm, deg_hbm, srcb, dstb,
              gb0, gb1, gb2, gb3, gb4, onesv, zbuf, dbuf, acc, accd,
              gsem, ssem, dsem):
    c = lax.axis_index("c")
    s = lax.axis_index("s")
    gb = [gb0, gb1, gb2, gb3, gb4]

    pltpu.sync_copy(ones16_hbm, onesv)
    pltpu.sync_copy(dst_hbm.at[s], dstb)

    for e in range(8):
        q, h = e // 2, e % 2

        @pl.when(c == e // 4)
        def _pass(q=q, h=h, e=e):
            pltpu.sync_copy(zrow_hbm, zbuf)
            for r in range(NROWS // ZCH):
                pltpu.sync_copy(
                    zbuf, acc.at[pl.ds(s * NROWS + r * ZCH, ZCH)])
            if e == 0:
                pltpu.sync_copy(z16_hbm, dbuf)
                for r in range(NROWS // ZCH):
                    pltpu.sync_copy(
                        dbuf, accd.at[pl.ds(s * NROWS + r * ZCH, ZCH)])

            pltpu.sync_copy((src0_hbm if h == 0 else src1_hbm).at[s], srcb)
            plsc.subcore_barrier()

            def group(gi, _):
                ds_ = [pltpu.async_copy(tok_hbm.at[q].at[srcb.at[gi * NB + b]],
                                        gb[b], gsem) for b in range(NB)]
                sd = []
                for b in range(NB):
                    ds_[b].wait()
                    sd.append(pltpu.async_copy(
                        gb[b], acc.at[dstb.at[gi * NB + b]], ssem, add=True))
                    if e == 0:
                        sd.append(pltpu.async_copy(
                            onesv, accd.at[dstb.at[gi * NB + b]], dsem,
                            add=True))
                for d in sd:
                    d.wait()
                return 0

            lax.fori_loop(0, NGRP, group, 0)
            plsc.subcore_barrier()
            for r in range(NROWS // ZCH):
                rs = pl.ds(s * NROWS + r * ZCH, ZCH)
                pltpu.sync_copy(acc.at[rs], zbuf)
                pltpu.sync_copy(zbuf, agg_hbm.at[e, rs])
                if e == 0:
                    pltpu.sync_copy(accd.at[rs], dbuf)
                    pltpu.sync_copy(dbuf, deg_hbm.at[rs])
            plsc.subcore_barrier()


def _agg(tok4, src, dst, zrow):
    mesh = plsc.VectorSubcoreMesh(
        core_axis_name="c", subcore_axis_name="s",
        num_cores=2, num_subcores=NTILES)
    k = pl.kernel(
        _agg_body,
        out_type=[
            jax.ShapeDtypeStruct((8, NP, 64), jnp.float32),
            jax.ShapeDtypeStruct((NP, 16), jnp.float32),
        ],
        mesh=mesh,
        scratch_types=[
            pltpu.VMEM((NCH, CHUNK), jnp.int32),
            pltpu.VMEM((NCH, CHUNK), jnp.int32),
            pltpu.VMEM((CHUNK, 64), jnp.float32),
            pltpu.VMEM((CHUNK, 64), jnp.float32),
            pltpu.VMEM((CHUNK, 64), jnp.float32),
            pltpu.VMEM((CHUNK, 64), jnp.float32),
            pltpu.VMEM((CHUNK, 64), jnp.float32),
            pltpu.VMEM((CHUNK, 16), jnp.float32),
            pltpu.VMEM((ZCH, 64), jnp.float32),
            pltpu.VMEM((ZCH, 16), jnp.float32),
            pltpu.VMEM_SHARED((NP, 64), jnp.float32),
            pltpu.VMEM_SHARED((NP, 16), jnp.float32),
            pltpu.SemaphoreType.DMA,
            pltpu.SemaphoreType.DMA,
            pltpu.SemaphoreType.DMA,
        ],
        compiler_params=pltpu.CompilerParams(use_tc_tiling_on_sc=False),
    )
    tok8v = tok4.reshape(4, 2 * NP, 64)
    src2 = src + src
    return k(tok8v, src2.reshape(NTILES, NCH, CHUNK),
             (src2 + 1).reshape(NTILES, NCH, CHUNK),
             dst.reshape(NTILES, NCH, CHUNK), zrow,
             jnp.zeros((ZCH, 16), jnp.float32),
             jnp.ones((CHUNK, 16), jnp.float32))


def _att_body(tok_ref, agg_ref, deg_ref, wq_ref, wk_ref, wv_ref,
              wo_ref, wout_ref, b_ref, out_ref):
    scale = 1.0 / jnp.maximum(deg_ref[:, 0:1], 1.0)
    bq = b_ref[0:1, :]
    bk = b_ref[1:2, :]
    bv = b_ref[2:3, :]
    bo = b_ref[3:4, :]
    bout = b_ref[4:5, :]

    q4, k4, v4 = [], [], []
    for qq in range(4):
        tq = tok_ref[qq]
        aq = jnp.concatenate(
            [agg_ref[2 * qq], agg_ref[2 * qq + 1]], axis=1) * scale
        q4.append(lax.dot(tq, wq_ref[...], preferred_element_type=jnp.float32) + bq)
        k4.append(lax.dot(aq, wk_ref[...], preferred_element_type=jnp.float32) + bk)
        v4.append(lax.dot(aq, wv_ref[...], preferred_element_type=jnp.float32) + bv)
    def vmajor(parts):
        return jnp.concatenate(
            [parts[v // 4][:, (v % 4) * D:(v % 4) * D + D] for v in range(V)],
            axis=0)

    q2 = vmajor(q4)
    k2 = vmajor(k4)
    v2 = vmajor(v4)

    rowg = lax.broadcasted_iota(jnp.int32, (ABV, ABV), 0) % AB
    colg = lax.broadcasted_iota(jnp.int32, (ABV, ABV), 1) % AB
    bias = jnp.where(rowg == colg, 0.0, NEG)

    ohs = []
    for h in range(H):
        qh = q2[:, h * DH:(h + 1) * DH] * 0.25
        kh = k2[:, h * DH:(h + 1) * DH]
        vh = v2[:, h * DH:(h + 1) * DH]
        s = lax.dot_general(qh, kh, (((1,), (1,)), ((), ())),
                            preferred_element_type=jnp.float32) + bias
        m = s.max(axis=1, keepdims=True)
        e = jnp.exp(s - m)
        rinv = 1.0 / e.sum(axis=1, keepdims=True)
        ohs.append(lax.dot(e, vh, preferred_element_type=jnp.float32) * rinv)
    o2 = jnp.concatenate(ohs, axis=1)

    conv = lax.dot(o2, wo_ref[...], preferred_element_type=jnp.float32) + bo
    conv = jnp.maximum(conv, 0.0)
    pooled = conv.reshape(V, AB, 128).mean(axis=0)
    logits = lax.dot(pooled[:, :D], wout_ref[...],
                     preferred_element_type=jnp.float32) + bout
    m2 = logits.max(axis=1, keepdims=True)
    lse = jnp.log(jnp.sum(jnp.exp(logits - m2), axis=1, keepdims=True))
    out_ref[...] = logits - m2 - lse


def _att(tok4, agg4, deg_b, wq_bd, wk_bd, wv_bd, wo_p, wout_p, b8):
    return pl.pallas_call(
        _att_body,
        grid=(NP // AB,),
        in_specs=[
            pl.BlockSpec((4, AB, 128), lambda i: (0, i, 0)),
            pl.BlockSpec((8, AB, 64), lambda i: (0, i, 0)),
            pl.BlockSpec((AB, 128), lambda i: (i, 0)),
            pl.BlockSpec((128, 128), lambda i: (0, 0)),
            pl.BlockSpec((128, 128), lambda i: (0, 0)),
            pl.BlockSpec((128, 128), lambda i: (0, 0)),
            pl.BlockSpec((D, 128), lambda i: (0, 0)),
            pl.BlockSpec((D, 128), lambda i: (0, 0)),
            pl.BlockSpec((8, 128), lambda i: (0, 0)),
        ],
        out_specs=pl.BlockSpec((AB, 128), lambda i: (i, 0)),
        out_shape=jax.ShapeDtypeStruct((NP, 128), jnp.float32),
    )(tok4, agg4, deg_b, wq_bd, wk_bd, wv_bd, wo_p, wout_p, b8)


def kernel(x, edge_index, emb, Wq, bq, Wk, bk, Wv, bv, Wo, bo, Wout, bout):
    x = x.astype(jnp.float32)
    xp = jnp.pad(x, ((0, NP - N), (0, NFP - NF)))
    u = jax.random.uniform(jax.random.key(42), (N, NF),
                           minval=1e-9, maxval=1.0)
    g = -jnp.log(-jnp.log(u))
    gp = jnp.pad(g, ((0, NP - N), (0, NFP - NF)), constant_values=NEG)
    embp = jnp.pad(emb.astype(jnp.float32), ((0, NFP - NF), (0, 128 - FEAT)))

    cnt8 = _colsum(xp)
    cntT = jnp.pad(cnt8.sum(axis=0)[:, None], ((0, 0), (0, 127)))
    tok4 = _tokens(xp, gp, cntT, embp)

    src = edge_index[0].astype(jnp.int32)
    dst = edge_index[1].astype(jnp.int32)
    zrow = jnp.zeros((ZCH, 64), jnp.float32)
    agg4, deg16 = _agg(tok4, src, dst, zrow)
    deg = deg16[:, 0]
    deg_b = jnp.broadcast_to(deg[:, None], (NP, 128))

    eye4 = jnp.eye(4, dtype=jnp.float32)
    wq_bd = jnp.kron(eye4, Wq.astype(jnp.float32))
    wk_bd = jnp.kron(eye4, Wk.astype(jnp.float32))
    wv_bd = jnp.kron(eye4, Wv.astype(jnp.float32))
    wo_p = jnp.pad(Wo.astype(jnp.float32), ((0, 0), (0, 128 - D)))
    wout_p = jnp.pad(Wout.astype(jnp.float32), ((0, 0), (0, 128 - OUT)))
    b8 = jnp.zeros((8, 128), jnp.float32)
    b8 = b8.at[0, :].set(jnp.tile(bq.astype(jnp.float32), 4))
    b8 = b8.at[1, :].set(jnp.tile(bk.astype(jnp.float32), 4))
    b8 = b8.at[2, :].set(jnp.tile(bv.astype(jnp.float32), 4))
    b8 = b8.at[3, :D].set(bo.astype(jnp.float32))
    b8 = b8.at[4, :].set(jnp.concatenate(
        [bout.astype(jnp.float32), jnp.full((128 - OUT,), NEG, jnp.float32)]))

    out = _att(tok4, agg4, deg_b, wq_bd, wk_bd, wv_bd, wo_p, wout_p, b8)
    return out[:N, :OUT]


# speedup vs baseline: 1.0995x; 1.0995x over previous
import functools

import jax
import jax.numpy as jnp
from jax import lax
from jax.experimental import pallas as pl
from jax.experimental.pallas import tpu as pltpu
from jax.experimental.pallas import tpu_sc as plsc

N = 10000
NF = 1433
V = 16
D = 32
FEAT = 31
H = 2
OUT = 7
E = 160000

NP = 10240
NFP = 1536
RB = 256
AB = 32
ABV = AB * V
DH = D // H
NEG = -1e30

NTILES = 16
NROWS = NP // NTILES
EPT = E // NTILES
CHUNK = 125
ZCH = 128


def _colsum_body(x_ref, out_ref):
    @pl.when(pl.program_id(0) == 0)
    def _init():
        out_ref[...] = jnp.zeros_like(out_ref)

    out_ref[...] += x_ref[...].reshape(RB // 8, 8, NFP).sum(axis=0)


def _colsum(xp):
    return pl.pallas_call(
        _colsum_body,
        grid=(NP // RB,),
        in_specs=[pl.BlockSpec((RB, NFP), lambda i: (i, 0))],
        out_specs=pl.BlockSpec((8, NFP), lambda i: (0, 0)),
        out_shape=jax.ShapeDtypeStruct((8, NFP), jnp.float32),
    )(xp)


def _tokens_body(x_ref, g_ref, cnt_ref, emb_ref, out_ref):
    xb = x_ref[...]
    gb = g_ref[...]

    cntc = cnt_ref[:, 0:1]
    p = cntc * (1.0 / N)
    sd = jnp.sqrt(p * (1.0 - p))
    sd1 = jnp.where(sd == 0.0, 1.0, sd)
    a0 = (0.0 - p) / sd1
    dd = 1.0 / sd1
    lane_t = lax.broadcasted_iota(jnp.int32, (NFP, 128), 1)
    embx = jnp.where(lane_t == FEAT, a0,
                     jnp.where(lane_t == FEAT + 1, dd, emb_ref[...]))
    embx_bf = embx.astype(jnp.bfloat16)

    npres = xb.sum(axis=1, keepdims=True)
    cpres = jnp.log(0.5 / jnp.maximum(npres, 1.0))
    cabs = jnp.log(0.5 / jnp.maximum(NF - npres, 1.0))
    scores = gb + jnp.where(xb != 0.0, cpres, cabs)

    colid = lax.broadcasted_iota(jnp.int32, (RB, NFP), 1)
    lane_o = lax.broadcasted_iota(jnp.int32, (RB, 128), 1)
    for t in range(V):
        m = scores.max(axis=1, keepdims=True)
        sc = jnp.min(jnp.where(scores == m, colid, NFP), axis=1, keepdims=True)
        selb = colid == sc
        sel = selb.astype(jnp.float32)
        row = lax.dot(selb.astype(jnp.bfloat16), embx_bf,
                      preferred_element_type=jnp.float32)
        pres = (sel * xb).sum(axis=1, keepdims=True)
        val = row[:, FEAT:FEAT + 1] + pres * row[:, FEAT + 1:FEAT + 2]
        tok = jnp.where(lane_o == FEAT, val, row)
        out_ref[t // 4, :, (t % 4) * D:(t % 4) * D + D] = tok[:, :D]
        scores = jnp.where(selb, NEG, scores)


def _tokens(xp, gp, cntT, embp):
    return pl.pallas_call(
        _tokens_body,
        grid=(NP // RB,),
        in_specs=[
            pl.BlockSpec((RB, NFP), lambda i: (i, 0)),
            pl.BlockSpec((RB, NFP), lambda i: (i, 0)),
            pl.BlockSpec((NFP, 128), lambda i: (0, 0)),
            pl.BlockSpec((NFP, 128), lambda i: (0, 0)),
        ],
        out_specs=pl.BlockSpec((4, RB, 128), lambda i: (0, i, 0)),
        out_shape=jax.ShapeDtypeStruct((4, NP, 128), jnp.float32),
    )(xp, gp, cntT, embp)


NCH = EPT // CHUNK
NB = 5
NGRP = NCH // NB


def _agg_body(tok_hbm, src0_hbm, src1_hbm, dst_hbm, zrow_hbm, z16_hbm,
              ones16_hbm, agg_hbm, deg_hbm, srcb, dstb,
              gb0, gb1, gb2, gb3, gb4, onesv, zbuf, dbuf, acc, accd,
              gsem, ssem, dsem):
    c = lax.axis_index("c")
    s = lax.axis_index("s")
    gb = [gb0, gb1, gb2, gb3, gb4]

    pltpu.sync_copy(ones16_hbm, onesv)
    pltpu.sync_copy(dst_hbm.at[s], dstb)

    for e in range(8):
        q, h = e // 2, e % 2

        @pl.when(c == e // 4)
        def _pass(q=q, h=h, e=e):
            pltpu.sync_copy(zrow_hbm, zbuf)
            for r in range(NROWS // ZCH):
                pltpu.sync_copy(
                    zbuf, acc.at[pl.ds(s * NROWS + r * ZCH, ZCH)])
            if e == 0:
                pltpu.sync_copy(z16_hbm, dbuf)
                for r in range(NROWS // ZCH):
                    pltpu.sync_copy(
                        dbuf, accd.at[pl.ds(s * NROWS + r * ZCH, ZCH)])

            pltpu.sync_copy((src0_hbm if h == 0 else src1_hbm).at[s], srcb)
            plsc.subcore_barrier()

            def group(gi, _):
                ds_ = [pltpu.async_copy(tok_hbm.at[q].at[srcb.at[gi * NB + b]],
                                        gb[b], gsem) for b in range(NB)]
                sd = []
                for b in range(NB):
                    ds_[b].wait()
                    sd.append(pltpu.async_copy(
                        gb[b], acc.at[dstb.at[gi * NB + b]], ssem, add=True))
                    if e == 0:
                        sd.append(pltpu.async_copy(
                            onesv, accd.at[dstb.at[gi * NB + b]], dsem,
                            add=True))
                for d in sd:
                    d.wait()
                return 0

            lax.fori_loop(0, NGRP, group, 0)
            plsc.subcore_barrier()
            for r in range(NROWS // ZCH):
                rs = pl.ds(s * NROWS + r * ZCH, ZCH)
                pltpu.sync_copy(acc.at[rs], zbuf)
                pltpu.sync_copy(zbuf, agg_hbm.at[e, rs])
                if e == 0:
                    pltpu.sync_copy(accd.at[rs], dbuf)
                    pltpu.sync_copy(dbuf, deg_hbm.at[rs])
            plsc.subcore_barrier()


def _agg(tok4, src, dst, zrow):
    mesh = plsc.VectorSubcoreMesh(
        core_axis_name="c", subcore_axis_name="s",
        num_cores=2, num_subcores=NTILES)
    k = pl.kernel(
        _agg_body,
        out_type=[
            jax.ShapeDtypeStruct((8, NP, 64), jnp.float32),
            jax.ShapeDtypeStruct((NP, 16), jnp.float32),
        ],
        mesh=mesh,
        scratch_types=[
            pltpu.VMEM((NCH, CHUNK), jnp.int32),
            pltpu.VMEM((NCH, CHUNK), jnp.int32),
            pltpu.VMEM((CHUNK, 64), jnp.float32),
            pltpu.VMEM((CHUNK, 64), jnp.float32),
            pltpu.VMEM((CHUNK, 64), jnp.float32),
            pltpu.VMEM((CHUNK, 64), jnp.float32),
            pltpu.VMEM((CHUNK, 64), jnp.float32),
            pltpu.VMEM((CHUNK, 16), jnp.float32),
            pltpu.VMEM((ZCH, 64), jnp.float32),
            pltpu.VMEM((ZCH, 16), jnp.float32),
            pltpu.VMEM_SHARED((NP, 64), jnp.float32),
            pltpu.VMEM_SHARED((NP, 16), jnp.float32),
            pltpu.SemaphoreType.DMA,
            pltpu.SemaphoreType.DMA,
            pltpu.SemaphoreType.DMA,
        ],
        compiler_params=pltpu.CompilerParams(use_tc_tiling_on_sc=False),
    )
    tok8v = tok4.reshape(4, 2 * NP, 64)
    src2 = src + src
    return k(tok8v, src2.reshape(NTILES, NCH, CHUNK),
             (src2 + 1).reshape(NTILES, NCH, CHUNK),
             dst.reshape(NTILES, NCH, CHUNK), zrow,
             jnp.zeros((ZCH, 16), jnp.float32),
             jnp.ones((CHUNK, 16), jnp.float32))


def _att_body(tok_ref, agg_ref, deg_ref, wq_ref, wk_ref, wv_ref,
              wo_ref, wout_ref, b_ref, out_ref):
    scale = 1.0 / jnp.maximum(deg_ref[:, 0:1], 1.0)
    bq = b_ref[0:1, :]
    bk = b_ref[1:2, :]
    bv = b_ref[2:3, :]
    bo = b_ref[3:4, :]
    bout = b_ref[4:5, :]

    q4, k4, v4 = [], [], []
    for qq in range(4):
        tq = tok_ref[qq]
        aq = jnp.concatenate(
            [agg_ref[2 * qq], agg_ref[2 * qq + 1]], axis=1) * scale
        q4.append(lax.dot(tq, wq_ref[...], preferred_element_type=jnp.float32) + bq)
        k4.append(lax.dot(aq, wk_ref[...], preferred_element_type=jnp.float32) + bk)
        v4.append(lax.dot(aq, wv_ref[...], preferred_element_type=jnp.float32) + bv)
    def vmajor(parts):
        return jnp.concatenate(
            [parts[v // 4][:, (v % 4) * D:(v % 4) * D + D] for v in range(V)],
            axis=0)

    q2 = vmajor(q4)
    k2 = vmajor(k4)
    v2 = vmajor(v4)

    rowg = lax.broadcasted_iota(jnp.int32, (ABV, ABV), 0) % AB
    colg = lax.broadcasted_iota(jnp.int32, (ABV, ABV), 1) % AB
    bias = jnp.where(rowg == colg, 0.0, NEG).astype(jnp.bfloat16)

    ones_bf = jnp.ones((ABV, 128), jnp.bfloat16)
    ohs = []
    for h in range(H):
        qh = (q2[:, h * DH:(h + 1) * DH] * 0.25).astype(jnp.bfloat16)
        kh = k2[:, h * DH:(h + 1) * DH].astype(jnp.bfloat16)
        vh = v2[:, h * DH:(h + 1) * DH].astype(jnp.bfloat16)
        s = lax.dot_general(qh, kh, (((1,), (1,)), ((), ())),
                            preferred_element_type=jnp.float32
                            ).astype(jnp.bfloat16) + bias
        m = s.max(axis=1, keepdims=True)
        e = jnp.exp(s - m)
        den = lax.dot(e, ones_bf,
                      preferred_element_type=jnp.float32)[:, 0:1]
        num = lax.dot(e, vh, preferred_element_type=jnp.float32)
        ohs.append(num / den)
    o2 = jnp.concatenate(ohs, axis=1)

    conv = lax.dot(o2, wo_ref[...], preferred_element_type=jnp.float32) + bo
    conv = jnp.maximum(conv, 0.0)
    pooled = conv.reshape(V, AB, 128).mean(axis=0)
    logits = lax.dot(pooled[:, :D], wout_ref[...],
                     preferred_element_type=jnp.float32) + bout
    m2 = logits.max(axis=1, keepdims=True)
    lse = jnp.log(jnp.sum(jnp.exp(logits - m2), axis=1, keepdims=True))
    out_ref[...] = logits - m2 - lse


def _att(tok4, agg4, deg_b, wq_bd, wk_bd, wv_bd, wo_p, wout_p, b8):
    return pl.pallas_call(
        _att_body,
        grid=(NP // AB,),
        in_specs=[
            pl.BlockSpec((4, AB, 128), lambda i: (0, i, 0)),
            pl.BlockSpec((8, AB, 64), lambda i: (0, i, 0)),
            pl.BlockSpec((AB, 128), lambda i: (i, 0)),
            pl.BlockSpec((128, 128), lambda i: (0, 0)),
            pl.BlockSpec((128, 128), lambda i: (0, 0)),
            pl.BlockSpec((128, 128), lambda i: (0, 0)),
            pl.BlockSpec((D, 128), lambda i: (0, 0)),
            pl.BlockSpec((D, 128), lambda i: (0, 0)),
            pl.BlockSpec((8, 128), lambda i: (0, 0)),
        ],
        out_specs=pl.BlockSpec((AB, 128), lambda i: (i, 0)),
        out_shape=jax.ShapeDtypeStruct((NP, 128), jnp.float32),
    )(tok4, agg4, deg_b, wq_bd, wk_bd, wv_bd, wo_p, wout_p, b8)


def kernel(x, edge_index, emb, Wq, bq, Wk, bk, Wv, bv, Wo, bo, Wout, bout):
    x = x.astype(jnp.float32)
    xp = jnp.pad(x, ((0, NP - N), (0, NFP - NF)))
    u = jax.random.uniform(jax.random.key(42), (N, NF),
                           minval=1e-9, maxval=1.0)
    g = -jnp.log(-jnp.log(u))
    gp = jnp.pad(g, ((0, NP - N), (0, NFP - NF)), constant_values=NEG)
    embp = jnp.pad(emb.astype(jnp.float32), ((0, NFP - NF), (0, 128 - FEAT)))

    cnt8 = _colsum(xp)
    cntT = jnp.pad(cnt8.sum(axis=0)[:, None], ((0, 0), (0, 127)))
    tok4 = _tokens(xp, gp, cntT, embp)

    src = edge_index[0].astype(jnp.int32)
    dst = edge_index[1].astype(jnp.int32)
    zrow = jnp.zeros((ZCH, 64), jnp.float32)
    agg4, deg16 = _agg(tok4, src, dst, zrow)
    deg = deg16[:, 0]
    deg_b = jnp.broadcast_to(deg[:, None], (NP, 128))

    eye4 = jnp.eye(4, dtype=jnp.float32)
    wq_bd = jnp.kron(eye4, Wq.astype(jnp.float32))
    wk_bd = jnp.kron(eye4, Wk.astype(jnp.float32))
    wv_bd = jnp.kron(eye4, Wv.astype(jnp.float32))
    wo_p = jnp.pad(Wo.astype(jnp.float32), ((0, 0), (0, 128 - D)))
    wout_p = jnp.pad(Wout.astype(jnp.float32), ((0, 0), (0, 128 - OUT)))
    b8 = jnp.zeros((8, 128), jnp.float32)
    b8 = b8.at[0, :].set(jnp.tile(bq.astype(jnp.float32), 4))
    b8 = b8.at[1, :].set(jnp.tile(bk.astype(jnp.float32), 4))
    b8 = b8.at[2, :].set(jnp.tile(bv.astype(jnp.float32), 4))
    b8 = b8.at[3, :D].set(bo.astype(jnp.float32))
    b8 = b8.at[4, :].set(jnp.concatenate(
        [bout.astype(jnp.float32), jnp.full((128 - OUT,), NEG, jnp.float32)]))

    out = _att(tok4, agg4, deg_b, wq_bd, wk_bd, wv_bd, wo_p, wout_p, b8)
    return out[:N, :OUT]
